# baseline (speedup 1.0000x reference)
import jax
import jax.numpy as jnp
from jax.experimental import pallas as pl
from jax.experimental.pallas import tpu as pltpu


def _mm_kernel(x_ref, w_ref, o_ref):
    o_ref[...] = jnp.dot(x_ref[...], w_ref[...],
                         preferred_element_type=jnp.float32)


def _prop_kernel(adj_ref, s_ref, b_ref, o_ref):
    acc = jnp.dot(adj_ref[...], s_ref[...],
                  preferred_element_type=jnp.float32)
    o_ref[...] = jnp.maximum(acc + b_ref[...], 0.0)


def _dec_kernel(za_ref, zb_ref, o_ref):
    p = jax.lax.dot_general(za_ref[...], zb_ref[...],
                            (((1,), (1,)), ((), ())),
                            preferred_element_type=jnp.float32)
    o_ref[...] = jax.nn.sigmoid(p)


def _support(inp, w):
    n, c = inp.shape[0], w.shape[1]
    return pl.pallas_call(
        _mm_kernel,
        out_shape=jax.ShapeDtypeStruct((n, c), jnp.float32),
    )(inp, w)


_BM_PROP = 400


def _propagate(adj, s, b):
    n = adj.shape[0]
    c = s.shape[1]
    return pl.pallas_call(
        _prop_kernel,
        grid=(n // _BM_PROP,),
        in_specs=[
            pl.BlockSpec((_BM_PROP, n), lambda i: (i, 0)),
            pl.BlockSpec((n, c), lambda i: (0, 0)),
            pl.BlockSpec((1, c), lambda i: (0, 0)),
        ],
        out_specs=pl.BlockSpec((_BM_PROP, c), lambda i: (i, 0)),
        out_shape=jax.ShapeDtypeStruct((n, c), jnp.float32),
        compiler_params=pltpu.CompilerParams(
            dimension_semantics=("parallel",)),
    )(adj, s, b)


_BM_DEC = 400


def _decode(z):
    n, k = z.shape
    return pl.pallas_call(
        _dec_kernel,
        grid=(n // _BM_DEC,),
        in_specs=[
            pl.BlockSpec((_BM_DEC, k), lambda i: (i, 0)),
            pl.BlockSpec((n, k), lambda i: (0, 0)),
        ],
        out_specs=pl.BlockSpec((_BM_DEC, n), lambda i: (i, 0)),
        out_shape=jax.ShapeDtypeStruct((n, n), jnp.float32),
        compiler_params=pltpu.CompilerParams(
            dimension_semantics=("parallel",)),
    )(z, z)


def kernel(x, adj, W0, b0, W1, b1, W2, b2):
    W12 = jnp.concatenate([W1, W2], axis=1)
    b12 = jnp.concatenate([b1, b2])[None, :]

    s0 = _support(x, W0)
    h = _propagate(adj, s0, b0[None, :])
    s12 = _support(h, W12)
    ml = _propagate(adj, s12, b12)
    mu = ml[:, :32]
    logvar = ml[:, 32:]
    z = mu
    adj_recon = _decode(z)
    return (adj_recon, z, mu, logvar)


# speedup vs baseline: 1.0436x; 1.0436x over previous
import jax
import jax.numpy as jnp
from jax.experimental import pallas as pl
from jax.experimental.pallas import tpu as pltpu


def _prop1_kernel(adj_ref, x_ref, w_ref, b_ref, o_ref, s_ref):
    @pl.when(pl.program_id(0) == 0)
    def _():
        s_ref[...] = jnp.dot(x_ref[...], w_ref[...],
                             preferred_element_type=jnp.float32)

    acc = jnp.dot(adj_ref[...], s_ref[...],
                  preferred_element_type=jnp.float32)
    o_ref[...] = jnp.maximum(acc + b_ref[...], 0.0)


def _prop2_kernel(adj_ref, h_ref, w_ref, b_ref, mu_ref, lv_ref, s_ref):
    @pl.when(pl.program_id(0) == 0)
    def _():
        s_ref[...] = jnp.dot(h_ref[...], w_ref[...],
                             preferred_element_type=jnp.float32)

    acc = jnp.dot(adj_ref[...], s_ref[...],
                  preferred_element_type=jnp.float32)
    acc = jnp.maximum(acc + b_ref[...], 0.0)
    mu_ref[...] = acc[:, :32]
    lv_ref[...] = acc[:, 32:]


def _dec_kernel(za_ref, zb_ref, o_ref):
    p = jax.lax.dot_general(za_ref[...], zb_ref[...],
                            (((1,), (1,)), ((), ())),
                            preferred_element_type=jnp.float32)
    o_ref[...] = jax.nn.sigmoid(p)


_BM = 400


def kernel(x, adj, W0, b0, W1, b1, W2, b2):
    n, nfeat = x.shape
    nhid = W0.shape[1]
    zdim = W1.shape[1]
    W12 = jnp.concatenate([W1, W2], axis=1)
    b12 = jnp.concatenate([b1, b2])[None, :]
    grid = (n // _BM,)
    seq = pltpu.CompilerParams(dimension_semantics=("arbitrary",))

    h = pl.pallas_call(
        _prop1_kernel,
        grid=grid,
        in_specs=[
            pl.BlockSpec((_BM, n), lambda i: (i, 0)),
            pl.BlockSpec((n, nfeat), lambda i: (0, 0)),
            pl.BlockSpec((nfeat, nhid), lambda i: (0, 0)),
            pl.BlockSpec((1, nhid), lambda i: (0, 0)),
        ],
        out_specs=pl.BlockSpec((_BM, nhid), lambda i: (i, 0)),
        out_shape=jax.ShapeDtypeStruct((n, nhid), jnp.float32),
        scratch_shapes=[pltpu.VMEM((n, nhid), jnp.float32)],
        compiler_params=seq,
    )(adj, x, W0, b0[None, :])

    mu, logvar = pl.pallas_call(
        _prop2_kernel,
        grid=grid,
        in_specs=[
            pl.BlockSpec((_BM, n), lambda i: (i, 0)),
            pl.BlockSpec((n, nhid), lambda i: (0, 0)),
            pl.BlockSpec((nhid, 2 * zdim), lambda i: (0, 0)),
            pl.BlockSpec((1, 2 * zdim), lambda i: (0, 0)),
        ],
        out_specs=[
            pl.BlockSpec((_BM, zdim), lambda i: (i, 0)),
            pl.BlockSpec((_BM, zdim), lambda i: (i, 0)),
        ],
        out_shape=[
            jax.ShapeDtypeStruct((n, zdim), jnp.float32),
            jax.ShapeDtypeStruct((n, zdim), jnp.float32),
        ],
        scratch_shapes=[pltpu.VMEM((n, 2 * zdim), jnp.float32)],
        compiler_params=seq,
    )(adj, h, W12, b12)

    adj_recon = pl.pallas_call(
        _dec_kernel,
        grid=grid,
        in_specs=[
            pl.BlockSpec((_BM, zdim), lambda i: (i, 0)),
            pl.BlockSpec((n, zdim), lambda i: (0, 0)),
        ],
        out_specs=pl.BlockSpec((_BM, n), lambda i: (i, 0)),
        out_shape=jax.ShapeDtypeStruct((n, n), jnp.float32),
        compiler_params=pltpu.CompilerParams(
            dimension_semantics=("parallel",)),
    )(mu, mu)

    return (adj_recon, mu, mu, logvar)


# speedup vs baseline: 1.0539x; 1.0099x over previous
import jax
import jax.numpy as jnp
from jax.experimental import pallas as pl
from jax.experimental.pallas import tpu as pltpu


def _prop1_kernel(adj_ref, x_ref, w_ref, b_ref, o_ref, s_ref):
    @pl.when(pl.program_id(0) == 0)
    def _():
        s_ref[...] = jnp.dot(x_ref[...], w_ref[...],
                             preferred_element_type=jnp.float32)

    acc = jnp.dot(adj_ref[...], s_ref[...],
                  preferred_element_type=jnp.float32)
    o_ref[...] = jnp.maximum(acc + b_ref[...], 0.0)


def _prop2_kernel(adj_ref, h_ref, w_ref, b_ref, mu_ref, lv_ref, s_ref):
    @pl.when(pl.program_id(0) == 0)
    def _():
        s_ref[...] = jnp.dot(h_ref[...], w_ref[...],
                             preferred_element_type=jnp.float32)

    acc = jnp.dot(adj_ref[...], s_ref[...],
                  preferred_element_type=jnp.float32)
    acc = jnp.maximum(acc + b_ref[...], 0.0)
    mu_ref[...] = acc[:, :32]
    lv_ref[...] = acc[:, 32:]


def _dec_kernel(za_ref, zb_ref, o_ref, zb_bf_ref):
    @pl.when(pl.program_id(0) == 0)
    def _():
        zb_bf_ref[...] = zb_ref[...].astype(jnp.bfloat16)

    p = jax.lax.dot_general(za_ref[...].astype(jnp.bfloat16), zb_bf_ref[...],
                            (((1,), (1,)), ((), ())),
                            preferred_element_type=jnp.float32)
    o_ref[...] = jax.nn.sigmoid(p)


_BM = 400


def kernel(x, adj, W0, b0, W1, b1, W2, b2):
    n, nfeat = x.shape
    nhid = W0.shape[1]
    zdim = W1.shape[1]
    W12 = jnp.concatenate([W1, W2], axis=1)
    b12 = jnp.concatenate([b1, b2])[None, :]
    grid = (n // _BM,)
    seq = pltpu.CompilerParams(dimension_semantics=("arbitrary",))

    h = pl.pallas_call(
        _prop1_kernel,
        grid=grid,
        in_specs=[
            pl.BlockSpec((_BM, n), lambda i: (i, 0)),
            pl.BlockSpec((n, nfeat), lambda i: (0, 0)),
            pl.BlockSpec((nfeat, nhid), lambda i: (0, 0)),
            pl.BlockSpec((1, nhid), lambda i: (0, 0)),
        ],
        out_specs=pl.BlockSpec((_BM, nhid), lambda i: (i, 0)),
        out_shape=jax.ShapeDtypeStruct((n, nhid), jnp.float32),
        scratch_shapes=[pltpu.VMEM((n, nhid), jnp.float32)],
        compiler_params=seq,
    )(adj, x, W0, b0[None, :])

    mu, logvar = pl.pallas_call(
        _prop2_kernel,
        grid=grid,
        in_specs=[
            pl.BlockSpec((_BM, n), lambda i: (i, 0)),
            pl.BlockSpec((n, nhid), lambda i: (0, 0)),
            pl.BlockSpec((nhid, 2 * zdim), lambda i: (0, 0)),
            pl.BlockSpec((1, 2 * zdim), lambda i: (0, 0)),
        ],
        out_specs=[
            pl.BlockSpec((_BM, zdim), lambda i: (i, 0)),
            pl.BlockSpec((_BM, zdim), lambda i: (i, 0)),
        ],
        out_shape=[
            jax.ShapeDtypeStruct((n, zdim), jnp.float32),
            jax.ShapeDtypeStruct((n, zdim), jnp.float32),
        ],
        scratch_shapes=[pltpu.VMEM((n, 2 * zdim), jnp.float32)],
        compiler_params=seq,
    )(adj, h, W12, b12)

    adj_recon = pl.pallas_call(
        _dec_kernel,
        grid=grid,
        in_specs=[
            pl.BlockSpec((_BM, zdim), lambda i: (i, 0)),
            pl.BlockSpec((n, zdim), lambda i: (0, 0)),
        ],
        out_specs=pl.BlockSpec((_BM, n), lambda i: (i, 0)),
        out_shape=jax.ShapeDtypeStruct((n, n), jnp.float32),
        scratch_shapes=[pltpu.VMEM((n, zdim), jnp.bfloat16)],
        compiler_params=seq,
    )(mu, mu)

    return (adj_recon, mu, mu, logvar)
